# baseline (speedup 1.0000x reference)
import functools

import jax
import jax.numpy as jnp
from jax import lax
from jax.experimental import pallas as pl
from jax.experimental.pallas import tpu as pltpu
from jax.experimental.pallas import tpu_sc as plsc

NC = 2
NS = 16
NW = NC * NS
LANES = 16


def _tree_sum(vs):
    while len(vs) > 1:
        vs = [vs[i] + vs[i + 1] for i in range(0, len(vs) - 1, 2)] + (
            [vs[-1]] if len(vs) % 2 else []
        )
    return vs[0]


@functools.partial(jax.jit, static_argnames=("vocab", "dim"))
def _relayout_table(tt, tail_rows, *, vocab, dim):
    strip = 512
    nstrip = vocab // strip
    tail_v = vocab - nstrip * strip
    swords = strip * dim

    mesh = plsc.VectorSubcoreMesh(core_axis_name="c", subcore_axis_name="s")

    @functools.partial(
        pl.kernel,
        out_type=jax.ShapeDtypeStruct((vocab * dim,), jnp.float32),
        mesh=mesh,
        compiler_params=pltpu.CompilerParams(
            needs_layout_passes=False, use_tc_tiling_on_sc=True
        ),
        scratch_types=[
            pltpu.VMEM((dim * strip,), jnp.float32),
            pltpu.VMEM((dim * strip,), jnp.float32),
            pltpu.VMEM((swords,), jnp.float32),
            pltpu.VMEM((swords,), jnp.float32),
            pltpu.VMEM((max(tail_v, 1), dim), jnp.float32),
            pltpu.SemaphoreType.DMA,
            pltpu.SemaphoreType.DMA,
            pltpu.SemaphoreType.DMA,
            pltpu.SemaphoreType.DMA,
        ],
    )
    def ka(tt_hbm, tail_hbm, out_hbm, in0, in1, ob0, ob1, tin, is0, is1,
           os0, os1):
        wid = lax.axis_index("s") * NC + lax.axis_index("c")
        s0 = wid * nstrip // NW
        s1 = (wid + 1) * nstrip // NW
        in_bufs = (in0, in1)
        out_bufs = (ob0, ob1)
        in_sems = (is0, is1)
        out_sems = (os0, os1)
        iota = lax.iota(jnp.int32, LANES)
        iota_s = iota * strip

        def fire_in(s, b):
            for e in range(dim):
                pltpu.async_copy(
                    tt_hbm.at[e, pl.ds(s * strip, strip)],
                    in_bufs[b].at[pl.ds(e * strip, strip)],
                    in_sems[b],
                )

        def drain_in(b):
            for e in range(dim):
                pltpu.make_async_copy(
                    tt_hbm.at[e, pl.ds(0, strip)],
                    in_bufs[b].at[pl.ds(e * strip, strip)],
                    in_sems[b],
                ).wait()

        def fire_out(s, b):
            pltpu.async_copy(
                out_bufs[b], out_hbm.at[pl.ds(s * swords, swords)], out_sems[b]
            )

        def drain_out(b):
            pltpu.make_async_copy(
                out_bufs[b], out_hbm.at[pl.ds(0, swords)], out_sems[b]
            ).wait()

        def transpose(b, ncol):
            in_f = in_bufs[b]
            out_f = out_bufs[b]

            def cbody(c, _):
                for eb in range(dim // LANES):
                    goff = eb * (LANES * strip) + c * LANES
                    soff = c * (LANES * dim) + eb * LANES
                    for d in range(LANES):
                        dg = (iota + d) & (LANES - 1)
                        val = plsc.load_gather(in_f, [iota_s + (dg + goff)])
                        plsc.store_scatter(
                            out_f, [dg * dim + (iota + soff)], val
                        )
                return 0

            lax.fori_loop(0, ncol // LANES, cbody, 0)

        fire_in(s0, 0)

        def sbody(s, _):
            i = s - s0
            for b in range(2):
                @pl.when((i & 1) == b)
                def _(b=b):
                    drain_in(b)

                    @pl.when(s + 1 < s1)
                    def _():
                        fire_in(s + 1, 1 - b)

                    @pl.when(i >= 2)
                    def _():
                        drain_out(b)

                    transpose(b, strip)
                    fire_out(s, b)
            return 0

        lax.fori_loop(s0, s1, sbody, 0)
        drain_out(0)
        drain_out(1)

        if tail_v:
            @pl.when(wid == NW - 1)
            def _():
                pltpu.sync_copy(tail_hbm, tin)

                def tb(v, _):
                    ob0[pl.ds(v * dim, LANES)] = tin[v, 0:16]
                    ob0[pl.ds(v * dim + LANES, LANES)] = tin[v, 16:32]
                    return 0

                lax.fori_loop(0, tail_v, tb, 0)
                pltpu.sync_copy(
                    ob0.at[pl.ds(0, tail_v * dim)],
                    out_hbm.at[pl.ds(nstrip * swords, tail_v * dim)],
                )

    return ka(tt, tail_rows)


@functools.partial(jax.jit, static_argnames=("batch", "hist", "dim"))
def _mean_emb(x_flat, table, *, batch, hist, dim):
    rw = batch // NW
    r_chunk = 4
    ng = rw // r_chunk
    sc = r_chunk * hist
    ch = 80
    nch = sc // ch
    kfull = hist // LANES
    tail = hist - kfull * LANES

    mesh = plsc.VectorSubcoreMesh(core_axis_name="c", subcore_axis_name="s")

    @functools.partial(
        pl.kernel,
        out_type=jax.ShapeDtypeStruct((batch, dim), jnp.float32),
        mesh=mesh,
        compiler_params=pltpu.CompilerParams(
            needs_layout_passes=False, use_tc_tiling_on_sc=False
        ),
        scratch_types=[
            pltpu.VMEM((rw * hist + (LANES - tail if tail else 0),), jnp.int32),
            pltpu.VMEM((sc, dim), jnp.float32),
            pltpu.VMEM((sc, dim), jnp.float32),
            pltpu.VMEM((rw, dim), jnp.float32),
            pltpu.SemaphoreType.DMA,
            pltpu.SemaphoreType.DMA,
        ],
    )
    def k(x_hbm, table_hbm, out_hbm, idx_v, rows0, rows1, out_v, sem0, sem1):
        wid = lax.axis_index("s") * NC + lax.axis_index("c")
        ibase = wid * (rw * hist)
        pltpu.sync_copy(x_hbm.at[pl.ds(ibase, rw * hist)], idx_v.at[pl.ds(0, rw * hist)])

        rows_bufs = (rows0, rows1)
        sems = (sem0, sem1)

        def fire(g, b):
            goff = g * sc
            for j in range(nch):
                pltpu.async_copy(
                    table_hbm.at[idx_v.at[pl.ds(goff + j * ch, ch)]],
                    rows_bufs[b].at[pl.ds(j * ch, ch)],
                    sems[b],
                )

        def drain(b):
            for j in range(nch):
                pltpu.make_async_copy(
                    table_hbm.at[idx_v.at[pl.ds(j * ch, ch)]],
                    rows_bufs[b].at[pl.ds(j * ch, ch)],
                    sems[b],
                ).wait()

        lane_lt_tail = lax.iota(jnp.int32, LANES) < tail

        def reduce(g, b):
            rows_v = rows_bufs[b]
            goff = g * sc
            for r in range(r_chunk):
                rbase = r * hist

                def kbody(kk, carry, rbase=rbase):
                    a0, a1, cnt = carry
                    o = rbase + kk * LANES
                    idx = idx_v[pl.ds(goff + o, LANES)]
                    cnt = cnt + plsc.all_reduce_population_count(idx != 0)
                    a0 = a0 + _tree_sum(
                        [rows_v[o + t, 0:16] for t in range(LANES)]
                    )
                    a1 = a1 + _tree_sum(
                        [rows_v[o + t, 16:32] for t in range(LANES)]
                    )
                    return a0, a1, cnt

                z = jnp.zeros((LANES,), jnp.float32)
                zi = jnp.zeros((LANES,), jnp.int32)
                a0, a1, cnt = lax.fori_loop(0, kfull, kbody, (z, z, zi))
                if tail:
                    o = rbase + kfull * LANES
                    idx = idx_v[pl.ds(goff + o, LANES)]
                    cnt = cnt + plsc.all_reduce_population_count(
                        (idx != 0) & lane_lt_tail
                    )
                    a0 = a0 + _tree_sum(
                        [rows_v[o + t, 0:16] for t in range(tail)]
                    )
                    a1 = a1 + _tree_sum(
                        [rows_v[o + t, 16:32] for t in range(tail)]
                    )
                inv = 1.0 / jnp.maximum(cnt.astype(jnp.float32), 1.0)
                row_out = g * r_chunk + r
                out_v[row_out, 0:16] = a0 * inv
                out_v[row_out, 16:32] = a1 * inv

        nt = ng // 2
        fire(0, 0)

        def tbody(t, _):
            g0 = t * 2
            fire(g0 + 1, 1)
            drain(0)
            reduce(g0, 0)

            @pl.when(t < nt - 1)
            def _():
                fire(g0 + 2, 0)

            drain(1)
            reduce(g0 + 1, 1)
            return 0

        lax.fori_loop(0, nt, tbody, 0)
        pltpu.sync_copy(out_v, out_hbm.at[pl.ds(wid * rw, rw)])

    return k(x_flat, table)


def kernel(x, table):
    batch, hist = x.shape
    vocab, dim = table.shape
    strip = 512
    tail_start = (vocab // strip) * strip
    tlin = _relayout_table(
        table.T, table[tail_start:], vocab=vocab, dim=dim
    )
    return _mean_emb(
        x.astype(jnp.int32).reshape(-1),
        tlin.reshape(vocab, dim),
        batch=batch,
        hist=hist,
        dim=dim,
    )


# speedup vs baseline: 1.2104x; 1.2104x over previous
import functools

import jax
import jax.numpy as jnp
from jax import lax
from jax.experimental import pallas as pl
from jax.experimental.pallas import tpu as pltpu
from jax.experimental.pallas import tpu_sc as plsc

NC = 2
NS = 16
NW = NC * NS
LANES = 16


def _tree_sum(vs):
    while len(vs) > 1:
        vs = [vs[i] + vs[i + 1] for i in range(0, len(vs) - 1, 2)] + (
            [vs[-1]] if len(vs) % 2 else []
        )
    return vs[0]


@functools.partial(jax.jit, static_argnames=("vocab", "dim"))
def _relayout_table(tt, tail_rows, *, vocab, dim):
    strip = 512
    nstrip = vocab // strip
    tail_v = vocab - nstrip * strip
    swords = strip * dim

    mesh = plsc.VectorSubcoreMesh(core_axis_name="c", subcore_axis_name="s")

    @functools.partial(
        pl.kernel,
        out_type=jax.ShapeDtypeStruct((vocab * dim,), jnp.float32),
        mesh=mesh,
        compiler_params=pltpu.CompilerParams(
            needs_layout_passes=False, use_tc_tiling_on_sc=True
        ),
        scratch_types=[
            pltpu.VMEM((dim * strip,), jnp.float32),
            pltpu.VMEM((dim * strip,), jnp.float32),
            pltpu.VMEM((swords,), jnp.float32),
            pltpu.VMEM((swords,), jnp.float32),
            pltpu.VMEM((max(tail_v, 1), dim), jnp.float32),
            pltpu.SemaphoreType.DMA,
            pltpu.SemaphoreType.DMA,
            pltpu.SemaphoreType.DMA,
            pltpu.SemaphoreType.DMA,
        ],
    )
    def ka(tt_hbm, tail_hbm, out_hbm, in0, in1, ob0, ob1, tin, is0, is1,
           os0, os1):
        wid = lax.axis_index("s") * NC + lax.axis_index("c")
        s0 = wid * nstrip // NW
        s1 = (wid + 1) * nstrip // NW
        in_bufs = (in0, in1)
        out_bufs = (ob0, ob1)
        in_sems = (is0, is1)
        out_sems = (os0, os1)
        iota = lax.iota(jnp.int32, LANES)
        iota_s = iota * strip

        def fire_in(s, b):
            for e in range(dim):
                pltpu.async_copy(
                    tt_hbm.at[e, pl.ds(s * strip, strip)],
                    in_bufs[b].at[pl.ds(e * strip, strip)],
                    in_sems[b],
                )

        def drain_in(b):
            for e in range(dim):
                pltpu.make_async_copy(
                    tt_hbm.at[e, pl.ds(0, strip)],
                    in_bufs[b].at[pl.ds(e * strip, strip)],
                    in_sems[b],
                ).wait()

        def fire_out(s, b):
            pltpu.async_copy(
                out_bufs[b], out_hbm.at[pl.ds(s * swords, swords)], out_sems[b]
            )

        def drain_out(b):
            pltpu.make_async_copy(
                out_bufs[b], out_hbm.at[pl.ds(0, swords)], out_sems[b]
            ).wait()

        def transpose(b, ncol):
            in_f = in_bufs[b]
            out_f = out_bufs[b]

            @plsc.parallel_loop(0, ncol // LANES, unroll=2)
            def _(c):
                for eb in range(dim // LANES):
                    goff = eb * (LANES * strip) + c * LANES
                    soff = c * (LANES * dim) + eb * LANES
                    for d in range(LANES):
                        dg = (iota + d) & (LANES - 1)
                        val = plsc.load_gather(in_f, [iota_s + (dg + goff)])
                        plsc.store_scatter(
                            out_f, [dg * dim + (iota + soff)], val
                        )

        fire_in(s0, 0)

        def sbody(s, _):
            i = s - s0
            for b in range(2):
                @pl.when((i & 1) == b)
                def _(b=b):
                    drain_in(b)

                    @pl.when(s + 1 < s1)
                    def _():
                        fire_in(s + 1, 1 - b)

                    @pl.when(i >= 2)
                    def _():
                        drain_out(b)

                    transpose(b, strip)
                    fire_out(s, b)
            return 0

        lax.fori_loop(s0, s1, sbody, 0)
        drain_out(0)
        drain_out(1)

        if tail_v:
            @pl.when(wid == NW - 1)
            def _():
                pltpu.sync_copy(tail_hbm, tin)

                def tb(v, _):
                    ob0[pl.ds(v * dim, LANES)] = tin[v, 0:16]
                    ob0[pl.ds(v * dim + LANES, LANES)] = tin[v, 16:32]
                    return 0

                lax.fori_loop(0, tail_v, tb, 0)
                pltpu.sync_copy(
                    ob0.at[pl.ds(0, tail_v * dim)],
                    out_hbm.at[pl.ds(nstrip * swords, tail_v * dim)],
                )

    return ka(tt, tail_rows)


@functools.partial(jax.jit, static_argnames=("batch", "hist", "dim"))
def _mean_emb(x_flat, table, *, batch, hist, dim):
    rw = batch // NW
    r_chunk = 4
    ng = rw // r_chunk
    sc = r_chunk * hist
    ch = 80
    nch = sc // ch
    kfull = hist // LANES
    tail = hist - kfull * LANES

    mesh = plsc.VectorSubcoreMesh(core_axis_name="c", subcore_axis_name="s")

    @functools.partial(
        pl.kernel,
        out_type=jax.ShapeDtypeStruct((batch, dim), jnp.float32),
        mesh=mesh,
        compiler_params=pltpu.CompilerParams(
            needs_layout_passes=False, use_tc_tiling_on_sc=False
        ),
        scratch_types=[
            pltpu.VMEM((rw * hist + (LANES - tail if tail else 0),), jnp.int32),
            pltpu.VMEM((sc, dim), jnp.float32),
            pltpu.VMEM((sc, dim), jnp.float32),
            pltpu.VMEM((rw, dim), jnp.float32),
            pltpu.SemaphoreType.DMA,
            pltpu.SemaphoreType.DMA,
        ],
    )
    def k(x_hbm, table_hbm, out_hbm, idx_v, rows0, rows1, out_v, sem0, sem1):
        wid = lax.axis_index("s") * NC + lax.axis_index("c")
        ibase = wid * (rw * hist)
        pltpu.sync_copy(x_hbm.at[pl.ds(ibase, rw * hist)], idx_v.at[pl.ds(0, rw * hist)])

        rows_bufs = (rows0, rows1)
        sems = (sem0, sem1)

        def fire(g, b):
            goff = g * sc
            for j in range(nch):
                pltpu.async_copy(
                    table_hbm.at[idx_v.at[pl.ds(goff + j * ch, ch)]],
                    rows_bufs[b].at[pl.ds(j * ch, ch)],
                    sems[b],
                )

        def drain(b):
            for j in range(nch):
                pltpu.make_async_copy(
                    table_hbm.at[idx_v.at[pl.ds(j * ch, ch)]],
                    rows_bufs[b].at[pl.ds(j * ch, ch)],
                    sems[b],
                ).wait()

        lane_lt_tail = lax.iota(jnp.int32, LANES) < tail

        def reduce(g, b):
            rows_v = rows_bufs[b]
            goff = g * sc
            for r in range(r_chunk):
                rbase = r * hist

                def kbody(kk, carry, rbase=rbase):
                    a0, a1, cnt = carry
                    o = rbase + kk * LANES
                    idx = idx_v[pl.ds(goff + o, LANES)]
                    cnt = cnt + plsc.all_reduce_population_count(idx != 0)
                    a0 = a0 + _tree_sum(
                        [rows_v[o + t, 0:16] for t in range(LANES)]
                    )
                    a1 = a1 + _tree_sum(
                        [rows_v[o + t, 16:32] for t in range(LANES)]
                    )
                    return a0, a1, cnt

                z = jnp.zeros((LANES,), jnp.float32)
                zi = jnp.zeros((LANES,), jnp.int32)
                a0, a1, cnt = lax.fori_loop(0, kfull, kbody, (z, z, zi))
                if tail:
                    o = rbase + kfull * LANES
                    idx = idx_v[pl.ds(goff + o, LANES)]
                    cnt = cnt + plsc.all_reduce_population_count(
                        (idx != 0) & lane_lt_tail
                    )
                    a0 = a0 + _tree_sum(
                        [rows_v[o + t, 0:16] for t in range(tail)]
                    )
                    a1 = a1 + _tree_sum(
                        [rows_v[o + t, 16:32] for t in range(tail)]
                    )
                inv = 1.0 / jnp.maximum(cnt.astype(jnp.float32), 1.0)
                row_out = g * r_chunk + r
                out_v[row_out, 0:16] = a0 * inv
                out_v[row_out, 16:32] = a1 * inv

        nt = ng // 2
        fire(0, 0)

        def tbody(t, _):
            g0 = t * 2
            fire(g0 + 1, 1)
            drain(0)
            reduce(g0, 0)

            @pl.when(t < nt - 1)
            def _():
                fire(g0 + 2, 0)

            drain(1)
            reduce(g0 + 1, 1)
            return 0

        lax.fori_loop(0, nt, tbody, 0)
        pltpu.sync_copy(out_v, out_hbm.at[pl.ds(wid * rw, rw)])

    return k(x_flat, table)


def kernel(x, table):
    batch, hist = x.shape
    vocab, dim = table.shape
    strip = 512
    tail_start = (vocab // strip) * strip
    tlin = _relayout_table(
        table.T, table[tail_start:], vocab=vocab, dim=dim
    )
    return _mean_emb(
        x.astype(jnp.int32).reshape(-1),
        tlin.reshape(vocab, dim),
        batch=batch,
        hist=hist,
        dim=dim,
    )


# speedup vs baseline: 1.5903x; 1.3139x over previous
import functools

import jax
import jax.numpy as jnp
from jax import lax
from jax.experimental import pallas as pl
from jax.experimental.pallas import tpu as pltpu
from jax.experimental.pallas import tpu_sc as plsc

NC = 2
NS = 16
NW = NC * NS
LANES = 16


def _tree_sum(vs):
    while len(vs) > 1:
        vs = [vs[i] + vs[i + 1] for i in range(0, len(vs) - 1, 2)] + (
            [vs[-1]] if len(vs) % 2 else []
        )
    return vs[0]


@functools.partial(jax.jit, static_argnames=("vocab", "dim"))
def _relayout_table(tt, tail_rows, *, vocab, dim):
    strip = 512
    nstrip = vocab // strip
    tail_v = vocab - nstrip * strip
    swords = strip * dim

    mesh = plsc.VectorSubcoreMesh(core_axis_name="c", subcore_axis_name="s")

    @functools.partial(
        pl.kernel,
        out_type=jax.ShapeDtypeStruct((vocab * dim,), jnp.float32),
        mesh=mesh,
        compiler_params=pltpu.CompilerParams(
            needs_layout_passes=False, use_tc_tiling_on_sc=True
        ),
        scratch_types=[
            pltpu.VMEM((dim * strip,), jnp.float32),
            pltpu.VMEM((dim * strip,), jnp.float32),
            pltpu.VMEM((swords,), jnp.float32),
            pltpu.VMEM((swords,), jnp.float32),
            pltpu.VMEM((max(tail_v, 1), dim), jnp.float32),
            pltpu.SemaphoreType.DMA,
            pltpu.SemaphoreType.DMA,
            pltpu.SemaphoreType.DMA,
            pltpu.SemaphoreType.DMA,
        ],
    )
    def ka(tt_hbm, tail_hbm, out_hbm, in0, in1, ob0, ob1, tin, is0, is1,
           os0, os1):
        wid = lax.axis_index("s") * NC + lax.axis_index("c")
        s0 = wid * nstrip // NW
        s1 = (wid + 1) * nstrip // NW
        in_bufs = (in0, in1)
        out_bufs = (ob0, ob1)
        in_sems = (is0, is1)
        out_sems = (os0, os1)
        iota = lax.iota(jnp.int32, LANES)
        iota_s = iota * strip

        def fire_in(s, b):
            for e in range(dim):
                pltpu.async_copy(
                    tt_hbm.at[e, pl.ds(s * strip, strip)],
                    in_bufs[b].at[pl.ds(e * strip, strip)],
                    in_sems[b],
                )

        def drain_in(b):
            for e in range(dim):
                pltpu.make_async_copy(
                    tt_hbm.at[e, pl.ds(0, strip)],
                    in_bufs[b].at[pl.ds(e * strip, strip)],
                    in_sems[b],
                ).wait()

        def fire_out(s, b):
            pltpu.async_copy(
                out_bufs[b], out_hbm.at[pl.ds(s * swords, swords)], out_sems[b]
            )

        def drain_out(b):
            pltpu.make_async_copy(
                out_bufs[b], out_hbm.at[pl.ds(0, swords)], out_sems[b]
            ).wait()

        def transpose(b, ncol):
            in_f = in_bufs[b]
            out_f = out_bufs[b]

            @plsc.parallel_loop(0, ncol // LANES, unroll=4)
            def _(c):
                for eb in range(dim // LANES):
                    goff = eb * (LANES * strip) + c * LANES
                    soff = c * (LANES * dim) + eb * LANES
                    for d in range(LANES):
                        dg = (iota + d) & (LANES - 1)
                        val = plsc.load_gather(in_f, [iota_s + (dg + goff)])
                        plsc.store_scatter(
                            out_f, [dg * dim + (iota + soff)], val
                        )

        fire_in(s0, 0)

        def sbody(s, _):
            i = s - s0
            for b in range(2):
                @pl.when((i & 1) == b)
                def _(b=b):
                    drain_in(b)

                    @pl.when(s + 1 < s1)
                    def _():
                        fire_in(s + 1, 1 - b)

                    @pl.when(i >= 2)
                    def _():
                        drain_out(b)

                    transpose(b, strip)
                    fire_out(s, b)
            return 0

        lax.fori_loop(s0, s1, sbody, 0)
        drain_out(0)
        drain_out(1)

        if tail_v:
            @pl.when(wid == NW - 1)
            def _():
                pltpu.sync_copy(tail_hbm, tin)

                def tb(v, _):
                    ob0[pl.ds(v * dim, LANES)] = tin[v, 0:16]
                    ob0[pl.ds(v * dim + LANES, LANES)] = tin[v, 16:32]
                    return 0

                lax.fori_loop(0, tail_v, tb, 0)
                pltpu.sync_copy(
                    ob0.at[pl.ds(0, tail_v * dim)],
                    out_hbm.at[pl.ds(nstrip * swords, tail_v * dim)],
                )

    return ka(tt, tail_rows)


@functools.partial(jax.jit, static_argnames=("batch", "hist", "dim"))
def _mean_emb(x_flat, table, *, batch, hist, dim):
    rw = batch // NW
    r_chunk = 4
    ng = rw // r_chunk
    sc = r_chunk * hist
    ch = 80
    nch = sc // ch
    kfull = hist // LANES
    tail = hist - kfull * LANES

    mesh = plsc.VectorSubcoreMesh(core_axis_name="c", subcore_axis_name="s")

    @functools.partial(
        pl.kernel,
        out_type=jax.ShapeDtypeStruct((batch, dim), jnp.float32),
        mesh=mesh,
        compiler_params=pltpu.CompilerParams(
            needs_layout_passes=False, use_tc_tiling_on_sc=False
        ),
        scratch_types=[
            pltpu.VMEM((rw * hist + (LANES - tail if tail else 0),), jnp.int32),
            pltpu.VMEM((sc, dim), jnp.float32),
            pltpu.VMEM((sc, dim), jnp.float32),
            pltpu.VMEM((rw, dim), jnp.float32),
            pltpu.SemaphoreType.DMA,
            pltpu.SemaphoreType.DMA,
        ],
    )
    def k(x_hbm, table_hbm, out_hbm, idx_v, rows0, rows1, out_v, sem0, sem1):
        wid = lax.axis_index("s") * NC + lax.axis_index("c")
        ibase = wid * (rw * hist)
        pltpu.sync_copy(x_hbm.at[pl.ds(ibase, rw * hist)], idx_v.at[pl.ds(0, rw * hist)])

        rows_bufs = (rows0, rows1)
        sems = (sem0, sem1)

        def fire(g, b):
            goff = g * sc
            for j in range(nch):
                pltpu.async_copy(
                    table_hbm.at[idx_v.at[pl.ds(goff + j * ch, ch)]],
                    rows_bufs[b].at[pl.ds(j * ch, ch)],
                    sems[b],
                )

        def drain(b):
            for j in range(nch):
                pltpu.make_async_copy(
                    table_hbm.at[idx_v.at[pl.ds(j * ch, ch)]],
                    rows_bufs[b].at[pl.ds(j * ch, ch)],
                    sems[b],
                ).wait()

        lane_lt_tail = lax.iota(jnp.int32, LANES) < tail

        def reduce(g, b):
            rows_v = rows_bufs[b]
            goff = g * sc
            for r in range(r_chunk):
                rbase = r * hist

                z = jnp.zeros((LANES,), jnp.float32)
                zi = jnp.zeros((LANES,), jnp.int32)

                @plsc.parallel_loop(0, kfull, carry=(z, z, zi))
                def acc(kk, carry, rbase=rbase):
                    a0, a1, cnt = carry
                    o = rbase + kk * LANES
                    idx = idx_v[pl.ds(goff + o, LANES)]
                    cnt = cnt + plsc.all_reduce_population_count(idx != 0)
                    a0 = a0 + _tree_sum(
                        [rows_v[o + t, 0:16] for t in range(LANES)]
                    )
                    a1 = a1 + _tree_sum(
                        [rows_v[o + t, 16:32] for t in range(LANES)]
                    )
                    return a0, a1, cnt

                a0, a1, cnt = acc
                if tail:
                    o = rbase + kfull * LANES
                    idx = idx_v[pl.ds(goff + o, LANES)]
                    cnt = cnt + plsc.all_reduce_population_count(
                        (idx != 0) & lane_lt_tail
                    )
                    a0 = a0 + _tree_sum(
                        [rows_v[o + t, 0:16] for t in range(tail)]
                    )
                    a1 = a1 + _tree_sum(
                        [rows_v[o + t, 16:32] for t in range(tail)]
                    )
                inv = 1.0 / jnp.maximum(cnt.astype(jnp.float32), 1.0)
                row_out = g * r_chunk + r
                out_v[row_out, 0:16] = a0 * inv
                out_v[row_out, 16:32] = a1 * inv

        nt = ng // 2
        fire(0, 0)

        def tbody(t, _):
            g0 = t * 2
            fire(g0 + 1, 1)
            drain(0)
            reduce(g0, 0)

            @pl.when(t < nt - 1)
            def _():
                fire(g0 + 2, 0)

            drain(1)
            reduce(g0 + 1, 1)
            return 0

        lax.fori_loop(0, nt, tbody, 0)
        pltpu.sync_copy(out_v, out_hbm.at[pl.ds(wid * rw, rw)])

    return k(x_flat, table)


def kernel(x, table):
    batch, hist = x.shape
    vocab, dim = table.shape
    strip = 512
    tail_start = (vocab // strip) * strip
    tlin = _relayout_table(
        table.T, table[tail_start:], vocab=vocab, dim=dim
    )
    return _mean_emb(
        x.astype(jnp.int32).reshape(-1),
        tlin.reshape(vocab, dim),
        batch=batch,
        hist=hist,
        dim=dim,
    )
